# baseline (speedup 1.0000x reference)
import functools

import jax
import jax.numpy as jnp
from jax import lax
from jax.experimental import pallas as pl
from jax.experimental.pallas import tpu as pltpu
from jax.experimental.pallas import tpu_sc as plsc

B, T, D = 2, 256, 1024
H = 16
FF = 4096
E = 8
EPS = 1e-5
N = B * T
NA = 2 * N
M = 256
R = 2816
FB = 512
NC, NS, L = 2, 16, 16
NW = NC * NS
TPW = N // NW


def _attn_body(x_ref, fb_ref, kw_ref, kb_ref, vw_ref, vb_ref, ow_ref, ob_ref,
               gw_ref, gb_ref, g1_ref, b1_ref, x1_ref, logits_ref):
    xb = x_ref[0]
    K = jnp.dot(xb, kw_ref[...], preferred_element_type=jnp.float32) + kb_ref[...]
    V = jnp.dot(xb, vw_ref[...], preferred_element_type=jnp.float32) + vb_ref[...]
    term1 = jnp.sum(K * V, axis=0, keepdims=True)
    wv = jnp.dot(fb_ref[0], V, preferred_element_type=jnp.float32) + term1
    attn = jnp.dot(wv, ow_ref[...], preferred_element_type=jnp.float32) + ob_ref[...]
    pre = xb + attn
    m = jnp.mean(pre, axis=1, keepdims=True)
    c = pre - m
    v = jnp.mean(c * c, axis=1, keepdims=True)
    x1 = c * lax.rsqrt(v + EPS) * g1_ref[...] + b1_ref[...]
    x1_ref[0] = x1
    logits_ref[0] = jnp.dot(x1, gw_ref[...], preferred_element_type=jnp.float32) + gb_ref[...]


def _attn_call(x, fb, kw, kb, vw, vb, ow, ob, gw, gb, g1, b1):
    full2 = lambda b: (0, 0)
    return pl.pallas_call(
        _attn_body,
        grid=(B,),
        in_specs=[
            pl.BlockSpec((1, T, D), lambda b: (b, 0, 0)),
            pl.BlockSpec((1, T, T), lambda b: (b, 0, 0)),
            pl.BlockSpec((D, D), full2),
            pl.BlockSpec((1, D), full2),
            pl.BlockSpec((D, D), full2),
            pl.BlockSpec((1, D), full2),
            pl.BlockSpec((D, D), full2),
            pl.BlockSpec((1, D), full2),
            pl.BlockSpec((D, E), full2),
            pl.BlockSpec((1, E), full2),
            pl.BlockSpec((1, D), full2),
            pl.BlockSpec((1, D), full2),
        ],
        out_specs=[
            pl.BlockSpec((1, T, D), lambda b: (b, 0, 0)),
            pl.BlockSpec((1, T, E), lambda b: (b, 0, 0)),
        ],
        out_shape=[
            jax.ShapeDtypeStruct((B, T, D), jnp.float32),
            jax.ShapeDtypeStruct((B, T, E), jnp.float32),
        ],
    )(x, fb, kw, kb, vw, vb, ow, ob, gw, gb, g1, b1)


def _route_body(logits_ref, pos_ref, sslot_ref, meta_ref):
    lg = logits_ref[...]
    mx = jnp.max(lg, axis=1, keepdims=True)
    ex = jnp.exp(lg - mx)
    sc = ex / jnp.sum(ex, axis=1, keepdims=True)
    io = lax.broadcasted_iota(jnp.int32, (N, E), 1)
    m1 = jnp.max(sc, axis=1, keepdims=True)
    e1 = jnp.min(jnp.where(sc == m1, io, E), axis=1, keepdims=True)
    sc2 = jnp.where(io == e1, -1.0, sc)
    m2 = jnp.max(sc2, axis=1, keepdims=True)
    e2 = jnp.min(jnp.where(sc2 == m2, io, E), axis=1, keepdims=True)
    ef = jnp.concatenate([e1, e2], axis=0)
    sf = jnp.concatenate([m1, m2], axis=0)
    ioA = lax.broadcasted_iota(jnp.int32, (NA, E), 1)
    oh = (ioA == ef).astype(jnp.float32)
    ri = lax.broadcasted_iota(jnp.int32, (NA, NA), 0)
    ci = lax.broadcasted_iota(jnp.int32, (NA, NA), 1)
    tril = (ci <= ri).astype(jnp.float32)
    cum = jnp.dot(tril, oh, preferred_element_type=jnp.float32)
    rank = jnp.sum(cum * oh, axis=1, keepdims=True) - 1.0
    counts = jnp.sum(oh, axis=0, keepdims=True)
    nbf = jnp.floor((counts + (M - 1)) / M)
    ui = lax.broadcasted_iota(jnp.int32, (E, E), 0)
    uj = lax.broadcasted_iota(jnp.int32, (E, E), 1)
    ux = (ui < uj).astype(jnp.float32)
    excl = jnp.dot(nbf, ux, preferred_element_type=jnp.float32)
    off = excl * M
    offsel = jnp.sum(oh * off, axis=1, keepdims=True)
    posi = (offsel + rank).astype(jnp.int32)
    pos_ref[...] = posi
    ioR = lax.broadcasted_iota(jnp.int32, (NA, R), 1)
    sslot_ref[...] = jnp.sum(jnp.where(ioR == posi, sf, 0.0), axis=0,
                             keepdims=True)
    meta_ref[...] = jnp.concatenate([nbf, off], axis=0).astype(jnp.int32)


def _route_call(logits):
    return pl.pallas_call(
        _route_body,
        out_shape=[
            jax.ShapeDtypeStruct((NA, 1), jnp.int32),
            jax.ShapeDtypeStruct((1, R), jnp.float32),
            jax.ShapeDtypeStruct((2, E), jnp.int32),
        ],
    )(logits)


@functools.cache
def _sc_mesh():
    return plsc.VectorSubcoreMesh(
        core_axis_name="c", subcore_axis_name="s",
        num_cores=NC, num_subcores=NS)


@functools.cache
def _sc_dispatch_kernel():
    return pl.kernel(
        _sc_dispatch_body,
        mesh=_sc_mesh(),
        out_type=jax.ShapeDtypeStruct((R, D), jnp.float32),
        scratch_types=[
            pltpu.VMEM((TPW, D), jnp.float32),
            pltpu.VMEM((TPW,), jnp.int32),
            pltpu.VMEM((TPW,), jnp.int32),
            pltpu.SemaphoreType.DMA,
        ],
    )


def _sc_dispatch_body(x1_hbm, pos_hbm, xg_hbm,
                      rows_v, idx0_v, idx1_v, sem):
    wid = lax.axis_index("s") * NC + lax.axis_index("c")
    base = wid * TPW
    pltpu.sync_copy(x1_hbm.at[pl.ds(base, TPW)], rows_v)
    pltpu.sync_copy(pos_hbm.at[pl.ds(base, TPW)], idx0_v)
    pltpu.sync_copy(pos_hbm.at[pl.ds(N + base, TPW)], idx1_v)
    pltpu.async_copy(rows_v, xg_hbm.at[idx0_v], sem).wait()
    pltpu.async_copy(rows_v, xg_hbm.at[idx1_v], sem).wait()


DB = 512
FB2 = 1024
NPA = D // DB
NPB = FF // FB2
HMAX = 512


def _ffn_body(nb_ref, off_ref, xg_ref, w1_ref, b1_ref, w2_ref, b2_ref, s_ref,
              yg_ref, h_ref):
    e = pl.program_id(0)
    p = pl.program_id(1)
    nb_e = nb_ref[e]
    off_e = off_ref[e]

    @pl.when(p < NPA)
    def _phase_a():
        d0 = pl.multiple_of(p * DB, DB)

        def body(i, carry):
            r0 = pl.multiple_of(off_e + i * M, M)
            l0 = pl.multiple_of(i * M, M)
            xb = xg_ref[pl.ds(r0, M), pl.ds(d0, DB)]
            acc = jnp.dot(xb, w1_ref[0], preferred_element_type=jnp.float32)
            prev = h_ref[pl.ds(l0, M), :].astype(jnp.float32)
            b1b = jnp.broadcast_to(b1_ref[0], acc.shape)
            h_ref[pl.ds(l0, M), :] = (
                acc + jnp.where(p == 0, b1b, prev)).astype(jnp.bfloat16)
            return carry

        lax.fori_loop(0, nb_e, body, 0)

    @pl.when(p >= NPA)
    def _phase_b():
        f0 = pl.multiple_of((p - NPA) * FB2, FB2)

        def body(i, carry):
            r0 = pl.multiple_of(off_e + i * M, M)
            l0 = pl.multiple_of(i * M, M)
            hb = jnp.maximum(h_ref[pl.ds(l0, M), pl.ds(f0, FB2)], 0.0)
            contrib = jnp.dot(hb.astype(jnp.float32), w2_ref[0],
                              preferred_element_type=jnp.float32)
            prev = yg_ref[pl.ds(r0, M), :]
            b2b = jnp.broadcast_to(b2_ref[0], contrib.shape)
            acc = contrib + jnp.where(p == NPA, b2b, prev)
            sb = s_ref[pl.ds(r0, M), :]
            acc = jnp.where(p == NPA + NPB - 1, acc * sb, acc)
            yg_ref[pl.ds(r0, M), :] = acc
            return carry

        lax.fori_loop(0, nb_e, body, 0)


def _ffn_call(nb, off, xg, e_w1, e_b1, e_w2, e_b2, sslot2d):
    grid_spec = pltpu.PrefetchScalarGridSpec(
        num_scalar_prefetch=2,
        grid=(E, NPA + NPB),
        in_specs=[
            pl.BlockSpec((R, D), lambda e, p, nb, off: (0, 0)),
            pl.BlockSpec((1, DB, FF),
                         lambda e, p, nb, off: (e, jnp.minimum(p, NPA - 1), 0)),
            pl.BlockSpec((1, 1, FF), lambda e, p, nb, off: (e, 0, 0)),
            pl.BlockSpec((1, FB2, D),
                         lambda e, p, nb, off: (e, jnp.maximum(p - NPA, 0), 0)),
            pl.BlockSpec((1, 1, D), lambda e, p, nb, off: (e, 0, 0)),
            pl.BlockSpec((R, 1), lambda e, p, nb, off: (0, 0)),
        ],
        out_specs=pl.BlockSpec((R, D), lambda e, p, nb, off: (0, 0)),
        scratch_shapes=[pltpu.VMEM((HMAX, FF), jnp.bfloat16)],
    )
    return pl.pallas_call(
        _ffn_body,
        grid_spec=grid_spec,
        out_shape=jax.ShapeDtypeStruct((R, D), jnp.float32),
    )(nb, off, xg, e_w1, e_b1.reshape(E, 1, FF), e_w2, e_b2.reshape(E, 1, D),
      sslot2d)


@functools.cache
def _sc_combine_kernel():
    return pl.kernel(
        _sc_combine_body,
        mesh=_sc_mesh(),
        out_type=jax.ShapeDtypeStruct((N, D), jnp.float32),
        scratch_types=[
            pltpu.VMEM((TPW, D), jnp.float32),
            pltpu.VMEM((TPW, D), jnp.float32),
            pltpu.VMEM((TPW, D), jnp.float32),
            pltpu.VMEM((TPW,), jnp.int32),
            pltpu.VMEM((TPW,), jnp.int32),
            pltpu.SemaphoreType.DMA,
        ],
    )


def _sc_combine_body(yg_hbm, pos_hbm, x1_hbm, res_hbm,
                     r0_v, r1_v, rx_v, idx0_v, idx1_v, sem):
    wid = lax.axis_index("s") * NC + lax.axis_index("c")
    base = wid * TPW
    pltpu.sync_copy(pos_hbm.at[pl.ds(base, TPW)], idx0_v)
    pltpu.sync_copy(pos_hbm.at[pl.ds(N + base, TPW)], idx1_v)
    pltpu.async_copy(yg_hbm.at[idx0_v], r0_v, sem).wait()
    pltpu.async_copy(yg_hbm.at[idx1_v], r1_v, sem).wait()
    pltpu.sync_copy(x1_hbm.at[pl.ds(base, TPW)], rx_v)

    UNROLL = 4
    CH = D // (UNROLL * L)

    def body(i, carry):
        t = i // CH
        d = i % CH
        for u in range(UNROLL):
            sl = pl.ds(d * UNROLL * L + u * L, L)
            r0_v[t, sl] = r0_v[t, sl] + r1_v[t, sl] + rx_v[t, sl]
        return carry

    lax.fori_loop(0, TPW * CH, body, 0)
    pltpu.sync_copy(r0_v, res_hbm.at[pl.ds(base, TPW)])


def _ln2_body(res_ref, g_ref, b_ref, out_ref):
    xb = res_ref[...]
    m = jnp.mean(xb, axis=1, keepdims=True)
    c = xb - m
    v = jnp.mean(c * c, axis=1, keepdims=True)
    out_ref[...] = c * lax.rsqrt(v + EPS) * g_ref[...] + b_ref[...]


def _ln2_call(res, g2, b2):
    return pl.pallas_call(
        _ln2_body,
        grid=(N // M,),
        in_specs=[
            pl.BlockSpec((M, D), lambda i: (i, 0)),
            pl.BlockSpec((1, D), lambda i: (0, 0)),
            pl.BlockSpec((1, D), lambda i: (0, 0)),
        ],
        out_specs=pl.BlockSpec((M, D), lambda i: (i, 0)),
        out_shape=jax.ShapeDtypeStruct((N, D), jnp.float32),
    )(res, g2, b2)


def kernel(x, fourier_bias, key_w, key_b, value_w, value_b, out_w, out_b,
           gate_w, gate_b, e_w1, e_b1, e_w2, e_b2, ln1_g, ln1_b, ln2_g, ln2_b):
    row = lambda v: v.reshape(1, -1)
    x1, logits = _attn_call(
        x, fourier_bias, key_w, row(key_b), value_w, row(value_b),
        out_w, row(out_b), gate_w, row(gate_b), row(ln1_g), row(ln1_b))
    x1_2d = x1.reshape(N, D)
    pos, sslot, meta = _route_call(logits.reshape(N, E))
    pos1 = pos.reshape(NA)
    xg = _sc_dispatch_kernel()(x1_2d, pos1)
    yg = _ffn_call(meta[0], meta[1], xg, e_w1, e_b1, e_w2, e_b2,
                   sslot.reshape(R, 1))
    res = _sc_combine_kernel()(yg, pos1, x1_2d)
    out = _ln2_call(res, row(ln2_g), row(ln2_b))
    return out.reshape(B, T, D)


# speedup vs baseline: 1.0302x; 1.0302x over previous
import functools

import jax
import jax.numpy as jnp
from jax import lax
from jax.experimental import pallas as pl
from jax.experimental.pallas import tpu as pltpu
from jax.experimental.pallas import tpu_sc as plsc

B, T, D = 2, 256, 1024
H = 16
FF = 4096
E = 8
EPS = 1e-5
N = B * T
NA = 2 * N
M = 128
R = 2048
FB = 512
NC, NS, L = 2, 16, 16
NW = NC * NS
TPW = N // NW


def _attn_body(x_ref, fb_ref, kw_ref, kb_ref, vw_ref, vb_ref, ow_ref, ob_ref,
               gw_ref, gb_ref, g1_ref, b1_ref, x1_ref, logits_ref):
    xb = x_ref[0]
    K = jnp.dot(xb, kw_ref[...], preferred_element_type=jnp.float32) + kb_ref[...]
    V = jnp.dot(xb, vw_ref[...], preferred_element_type=jnp.float32) + vb_ref[...]
    term1 = jnp.sum(K * V, axis=0, keepdims=True)
    wv = jnp.dot(fb_ref[0], V, preferred_element_type=jnp.float32) + term1
    attn = jnp.dot(wv, ow_ref[...], preferred_element_type=jnp.float32) + ob_ref[...]
    pre = xb + attn
    m = jnp.mean(pre, axis=1, keepdims=True)
    c = pre - m
    v = jnp.mean(c * c, axis=1, keepdims=True)
    x1 = c * lax.rsqrt(v + EPS) * g1_ref[...] + b1_ref[...]
    x1_ref[0] = x1
    logits_ref[0] = jnp.dot(x1, gw_ref[...], preferred_element_type=jnp.float32) + gb_ref[...]


def _attn_call(x, fb, kw, kb, vw, vb, ow, ob, gw, gb, g1, b1):
    full2 = lambda b: (0, 0)
    return pl.pallas_call(
        _attn_body,
        grid=(B,),
        in_specs=[
            pl.BlockSpec((1, T, D), lambda b: (b, 0, 0)),
            pl.BlockSpec((1, T, T), lambda b: (b, 0, 0)),
            pl.BlockSpec((D, D), full2),
            pl.BlockSpec((1, D), full2),
            pl.BlockSpec((D, D), full2),
            pl.BlockSpec((1, D), full2),
            pl.BlockSpec((D, D), full2),
            pl.BlockSpec((1, D), full2),
            pl.BlockSpec((D, E), full2),
            pl.BlockSpec((1, E), full2),
            pl.BlockSpec((1, D), full2),
            pl.BlockSpec((1, D), full2),
        ],
        out_specs=[
            pl.BlockSpec((1, T, D), lambda b: (b, 0, 0)),
            pl.BlockSpec((1, T, E), lambda b: (b, 0, 0)),
        ],
        out_shape=[
            jax.ShapeDtypeStruct((B, T, D), jnp.float32),
            jax.ShapeDtypeStruct((B, T, E), jnp.float32),
        ],
    )(x, fb, kw, kb, vw, vb, ow, ob, gw, gb, g1, b1)


def _route_body(logits_ref, pos_ref, sslot_ref, meta_ref):
    lg = logits_ref[...]
    mx = jnp.max(lg, axis=1, keepdims=True)
    ex = jnp.exp(lg - mx)
    sc = ex / jnp.sum(ex, axis=1, keepdims=True)
    io = lax.broadcasted_iota(jnp.int32, (N, E), 1)
    m1 = jnp.max(sc, axis=1, keepdims=True)
    e1 = jnp.min(jnp.where(sc == m1, io, E), axis=1, keepdims=True)
    sc2 = jnp.where(io == e1, -1.0, sc)
    m2 = jnp.max(sc2, axis=1, keepdims=True)
    e2 = jnp.min(jnp.where(sc2 == m2, io, E), axis=1, keepdims=True)
    ef = jnp.concatenate([e1, e2], axis=0)
    sf = jnp.concatenate([m1, m2], axis=0)
    ioA = lax.broadcasted_iota(jnp.int32, (NA, E), 1)
    oh = (ioA == ef).astype(jnp.float32)
    ri = lax.broadcasted_iota(jnp.int32, (NA, NA), 0)
    ci = lax.broadcasted_iota(jnp.int32, (NA, NA), 1)
    tril = (ci <= ri).astype(jnp.float32)
    cum = jnp.dot(tril, oh, preferred_element_type=jnp.float32)
    rank = jnp.sum(cum * oh, axis=1, keepdims=True) - 1.0
    counts = jnp.sum(oh, axis=0, keepdims=True)
    nbf = jnp.floor((counts + (M - 1)) / M)
    ui = lax.broadcasted_iota(jnp.int32, (E, E), 0)
    uj = lax.broadcasted_iota(jnp.int32, (E, E), 1)
    ux = (ui < uj).astype(jnp.float32)
    excl = jnp.dot(nbf, ux, preferred_element_type=jnp.float32)
    off = excl * M
    offsel = jnp.sum(oh * off, axis=1, keepdims=True)
    posi = (offsel + rank).astype(jnp.int32)
    pos_ref[...] = posi
    ioR = lax.broadcasted_iota(jnp.int32, (NA, R), 1)
    sslot_ref[...] = jnp.sum(jnp.where(ioR == posi, sf, 0.0), axis=0,
                             keepdims=True)
    meta_ref[...] = jnp.concatenate([nbf, off], axis=0).astype(jnp.int32)


def _route_call(logits):
    return pl.pallas_call(
        _route_body,
        out_shape=[
            jax.ShapeDtypeStruct((NA, 1), jnp.int32),
            jax.ShapeDtypeStruct((1, R), jnp.float32),
            jax.ShapeDtypeStruct((2, E), jnp.int32),
        ],
    )(logits)


@functools.cache
def _sc_mesh():
    return plsc.VectorSubcoreMesh(
        core_axis_name="c", subcore_axis_name="s",
        num_cores=NC, num_subcores=NS)


@functools.cache
def _sc_dispatch_kernel():
    return pl.kernel(
        _sc_dispatch_body,
        mesh=_sc_mesh(),
        out_type=jax.ShapeDtypeStruct((R, D), jnp.float32),
        scratch_types=[
            pltpu.VMEM((TPW, D), jnp.float32),
            pltpu.VMEM((TPW,), jnp.int32),
            pltpu.VMEM((TPW,), jnp.int32),
            pltpu.SemaphoreType.DMA,
        ],
    )


def _sc_dispatch_body(x1_hbm, pos_hbm, xg_hbm,
                      rows_v, idx0_v, idx1_v, sem):
    wid = lax.axis_index("s") * NC + lax.axis_index("c")
    base = wid * TPW
    pltpu.sync_copy(x1_hbm.at[pl.ds(base, TPW)], rows_v)
    pltpu.sync_copy(pos_hbm.at[pl.ds(base, TPW)], idx0_v)
    pltpu.sync_copy(pos_hbm.at[pl.ds(N + base, TPW)], idx1_v)
    pltpu.async_copy(rows_v, xg_hbm.at[idx0_v], sem).wait()
    pltpu.async_copy(rows_v, xg_hbm.at[idx1_v], sem).wait()


DB = 512
FB2 = 2048
NPA = D // DB
NPB = FF // FB2
HMAX = 512


def _ffn_body(nb_ref, off_ref, xg_ref, w1_ref, b1_ref, w2_ref, b2_ref, s_ref,
              yg_ref, h_ref):
    e = pl.program_id(0)
    p = pl.program_id(1)
    nb_e = nb_ref[e]
    off_e = off_ref[e]

    @pl.when(p < NPA)
    def _phase_a():
        d0 = pl.multiple_of(p * DB, DB)

        def body(i, carry):
            r0 = pl.multiple_of(off_e + i * M, M)
            l0 = pl.multiple_of(i * M, M)
            xb = xg_ref[pl.ds(r0, M), pl.ds(d0, DB)]
            acc = jnp.dot(xb, w1_ref[0], preferred_element_type=jnp.float32,
                          precision=lax.Precision.DEFAULT)
            prev = h_ref[pl.ds(l0, M), :].astype(jnp.float32)
            b1b = jnp.broadcast_to(b1_ref[0], acc.shape)
            h_ref[pl.ds(l0, M), :] = (
                acc + jnp.where(p == 0, b1b, prev)).astype(jnp.bfloat16)
            return carry

        lax.fori_loop(0, nb_e, body, 0)

    @pl.when(p >= NPA)
    def _phase_b():
        f0 = pl.multiple_of((p - NPA) * FB2, FB2)

        def body(i, carry):
            r0 = pl.multiple_of(off_e + i * M, M)
            l0 = pl.multiple_of(i * M, M)
            hb = jnp.maximum(h_ref[pl.ds(l0, M), pl.ds(f0, FB2)], 0.0)
            contrib = jnp.dot(hb.astype(jnp.float32), w2_ref[0],
                              preferred_element_type=jnp.float32,
                              precision=lax.Precision.DEFAULT)
            prev = yg_ref[pl.ds(r0, M), :]
            b2b = jnp.broadcast_to(b2_ref[0], contrib.shape)
            acc = contrib + jnp.where(p == NPA, b2b, prev)
            sb = s_ref[pl.ds(r0, M), :]
            acc = jnp.where(p == NPA + NPB - 1, acc * sb, acc)
            yg_ref[pl.ds(r0, M), :] = acc
            return carry

        lax.fori_loop(0, nb_e, body, 0)


def _ffn_call(nb, off, xg, e_w1, e_b1, e_w2, e_b2, sslot2d):
    grid_spec = pltpu.PrefetchScalarGridSpec(
        num_scalar_prefetch=2,
        grid=(E, NPA + NPB),
        in_specs=[
            pl.BlockSpec((R, D), lambda e, p, nb, off: (0, 0)),
            pl.BlockSpec((1, DB, FF),
                         lambda e, p, nb, off: (e, jnp.minimum(p, NPA - 1), 0)),
            pl.BlockSpec((1, 1, FF), lambda e, p, nb, off: (e, 0, 0)),
            pl.BlockSpec((1, FB2, D),
                         lambda e, p, nb, off: (e, jnp.maximum(p - NPA, 0), 0)),
            pl.BlockSpec((1, 1, D), lambda e, p, nb, off: (e, 0, 0)),
            pl.BlockSpec((R, 1), lambda e, p, nb, off: (0, 0)),
        ],
        out_specs=pl.BlockSpec((R, D), lambda e, p, nb, off: (0, 0)),
        scratch_shapes=[pltpu.VMEM((HMAX, FF), jnp.bfloat16)],
    )
    return pl.pallas_call(
        _ffn_body,
        grid_spec=grid_spec,
        out_shape=jax.ShapeDtypeStruct((R, D), jnp.float32),
    )(nb, off, xg, e_w1, e_b1.reshape(E, 1, FF), e_w2, e_b2.reshape(E, 1, D),
      sslot2d)


@functools.cache
def _sc_combine_kernel():
    return pl.kernel(
        _sc_combine_body,
        mesh=_sc_mesh(),
        out_type=jax.ShapeDtypeStruct((N, D), jnp.float32),
        scratch_types=[
            pltpu.VMEM((TPW, D), jnp.float32),
            pltpu.VMEM((TPW, D), jnp.float32),
            pltpu.VMEM((TPW, D), jnp.float32),
            pltpu.VMEM((TPW,), jnp.int32),
            pltpu.VMEM((TPW,), jnp.int32),
            pltpu.SemaphoreType.DMA,
        ],
    )


def _sc_combine_body(yg_hbm, pos_hbm, x1_hbm, res_hbm,
                     r0_v, r1_v, rx_v, idx0_v, idx1_v, sem):
    wid = lax.axis_index("s") * NC + lax.axis_index("c")
    base = wid * TPW
    pltpu.sync_copy(pos_hbm.at[pl.ds(base, TPW)], idx0_v)
    pltpu.sync_copy(pos_hbm.at[pl.ds(N + base, TPW)], idx1_v)
    pltpu.async_copy(yg_hbm.at[idx0_v], r0_v, sem).wait()
    pltpu.async_copy(yg_hbm.at[idx1_v], r1_v, sem).wait()
    pltpu.sync_copy(x1_hbm.at[pl.ds(base, TPW)], rx_v)

    UNROLL = 4
    CH = D // (UNROLL * L)

    def body(i, carry):
        t = i // CH
        d = i % CH
        for u in range(UNROLL):
            sl = pl.ds(d * UNROLL * L + u * L, L)
            r0_v[t, sl] = r0_v[t, sl] + r1_v[t, sl] + rx_v[t, sl]
        return carry

    lax.fori_loop(0, TPW * CH, body, 0)
    pltpu.sync_copy(r0_v, res_hbm.at[pl.ds(base, TPW)])


def _ln2_body(res_ref, g_ref, b_ref, out_ref):
    xb = res_ref[...]
    m = jnp.mean(xb, axis=1, keepdims=True)
    c = xb - m
    v = jnp.mean(c * c, axis=1, keepdims=True)
    out_ref[...] = c * lax.rsqrt(v + EPS) * g_ref[...] + b_ref[...]


def _ln2_call(res, g2, b2):
    return pl.pallas_call(
        _ln2_body,
        grid=(N // M,),
        in_specs=[
            pl.BlockSpec((M, D), lambda i: (i, 0)),
            pl.BlockSpec((1, D), lambda i: (0, 0)),
            pl.BlockSpec((1, D), lambda i: (0, 0)),
        ],
        out_specs=pl.BlockSpec((M, D), lambda i: (i, 0)),
        out_shape=jax.ShapeDtypeStruct((N, D), jnp.float32),
    )(res, g2, b2)


def kernel(x, fourier_bias, key_w, key_b, value_w, value_b, out_w, out_b,
           gate_w, gate_b, e_w1, e_b1, e_w2, e_b2, ln1_g, ln1_b, ln2_g, ln2_b):
    row = lambda v: v.reshape(1, -1)
    x1, logits = _attn_call(
        x, fourier_bias, key_w, row(key_b), value_w, row(value_b),
        out_w, row(out_b), gate_w, row(gate_b), row(ln1_g), row(ln1_b))
    x1_2d = x1.reshape(N, D)
    pos, sslot, meta = _route_call(logits.reshape(N, E))
    pos1 = pos.reshape(NA)
    xg = _sc_dispatch_kernel()(x1_2d, pos1)
    yg = _ffn_call(meta[0], meta[1], xg, e_w1, e_b1, e_w2, e_b2,
                   sslot.reshape(R, 1))
    res = _sc_combine_kernel()(yg, pos1, x1_2d)
    out = _ln2_call(res, row(ln2_g), row(ln2_b))
    return out.reshape(B, T, D)


# speedup vs baseline: 1.0404x; 1.0099x over previous
import functools

import jax
import jax.numpy as jnp
from jax import lax
from jax.experimental import pallas as pl
from jax.experimental.pallas import tpu as pltpu
from jax.experimental.pallas import tpu_sc as plsc

B, T, D = 2, 256, 1024
H = 16
FF = 4096
E = 8
EPS = 1e-5
N = B * T
NA = 2 * N
M = 128
R = 2048
FB = 512
NC, NS, L = 2, 16, 16
NW = NC * NS
TPW = N // NW


def _attn_body(x_ref, fb_ref, kw_ref, kb_ref, vw_ref, vb_ref, ow_ref, ob_ref,
               gw_ref, gb_ref, g1_ref, b1_ref, x1_ref, logits_ref):
    xb = x_ref[0]
    K = jnp.dot(xb, kw_ref[...], preferred_element_type=jnp.float32) + kb_ref[...]
    V = jnp.dot(xb, vw_ref[...], preferred_element_type=jnp.float32) + vb_ref[...]
    term1 = jnp.sum(K * V, axis=0, keepdims=True)
    wv = jnp.dot(fb_ref[0], V, preferred_element_type=jnp.float32) + term1
    attn = jnp.dot(wv, ow_ref[...], preferred_element_type=jnp.float32) + ob_ref[...]
    pre = xb + attn
    m = jnp.mean(pre, axis=1, keepdims=True)
    c = pre - m
    v = jnp.mean(c * c, axis=1, keepdims=True)
    x1 = c * lax.rsqrt(v + EPS) * g1_ref[...] + b1_ref[...]
    x1_ref[0] = x1
    logits_ref[0] = jnp.dot(x1, gw_ref[...], preferred_element_type=jnp.float32) + gb_ref[...]


def _attn_call(x, fb, kw, kb, vw, vb, ow, ob, gw, gb, g1, b1):
    full2 = lambda b: (0, 0)
    return pl.pallas_call(
        _attn_body,
        grid=(B,),
        in_specs=[
            pl.BlockSpec((1, T, D), lambda b: (b, 0, 0)),
            pl.BlockSpec((1, T, T), lambda b: (b, 0, 0)),
            pl.BlockSpec((D, D), full2),
            pl.BlockSpec((1, D), full2),
            pl.BlockSpec((D, D), full2),
            pl.BlockSpec((1, D), full2),
            pl.BlockSpec((D, D), full2),
            pl.BlockSpec((1, D), full2),
            pl.BlockSpec((D, E), full2),
            pl.BlockSpec((1, E), full2),
            pl.BlockSpec((1, D), full2),
            pl.BlockSpec((1, D), full2),
        ],
        out_specs=[
            pl.BlockSpec((1, T, D), lambda b: (b, 0, 0)),
            pl.BlockSpec((1, T, E), lambda b: (b, 0, 0)),
        ],
        out_shape=[
            jax.ShapeDtypeStruct((B, T, D), jnp.float32),
            jax.ShapeDtypeStruct((B, T, E), jnp.float32),
        ],
    )(x, fb, kw, kb, vw, vb, ow, ob, gw, gb, g1, b1)


def _route_body(logits_ref, pos_ref, sslot_ref, meta_ref):
    lg = logits_ref[...]
    mx = jnp.max(lg, axis=1, keepdims=True)
    ex = jnp.exp(lg - mx)
    sc = ex / jnp.sum(ex, axis=1, keepdims=True)
    io = lax.broadcasted_iota(jnp.int32, (N, E), 1)
    m1 = jnp.max(sc, axis=1, keepdims=True)
    e1 = jnp.min(jnp.where(sc == m1, io, E), axis=1, keepdims=True)
    sc2 = jnp.where(io == e1, -1.0, sc)
    m2 = jnp.max(sc2, axis=1, keepdims=True)
    e2 = jnp.min(jnp.where(sc2 == m2, io, E), axis=1, keepdims=True)
    ef = jnp.concatenate([e1, e2], axis=0)
    sf = jnp.concatenate([m1, m2], axis=0)
    ioA = lax.broadcasted_iota(jnp.int32, (NA, E), 1)
    oh = (ioA == ef).astype(jnp.float32)
    ri = lax.broadcasted_iota(jnp.int32, (NA, NA), 0)
    ci = lax.broadcasted_iota(jnp.int32, (NA, NA), 1)
    tril = (ci <= ri).astype(jnp.float32)
    cum = jnp.dot(tril, oh, preferred_element_type=jnp.float32)
    rank = jnp.sum(cum * oh, axis=1, keepdims=True) - 1.0
    counts = jnp.sum(oh, axis=0, keepdims=True)
    nbf = jnp.floor((counts + (M - 1)) / M)
    ui = lax.broadcasted_iota(jnp.int32, (E, E), 0)
    uj = lax.broadcasted_iota(jnp.int32, (E, E), 1)
    ux = (ui < uj).astype(jnp.float32)
    excl = jnp.dot(nbf, ux, preferred_element_type=jnp.float32)
    off = excl * M
    offsel = jnp.sum(oh * off, axis=1, keepdims=True)
    posi = (offsel + rank).astype(jnp.int32)
    pos_ref[...] = posi
    ioR = lax.broadcasted_iota(jnp.int32, (NA, R), 1)
    sslot_ref[...] = jnp.sum(jnp.where(ioR == posi, sf, 0.0), axis=0,
                             keepdims=True)
    meta_ref[...] = jnp.concatenate([nbf, off], axis=0).astype(jnp.int32)


def _route_call(logits):
    return pl.pallas_call(
        _route_body,
        out_shape=[
            jax.ShapeDtypeStruct((NA, 1), jnp.int32),
            jax.ShapeDtypeStruct((1, R), jnp.float32),
            jax.ShapeDtypeStruct((2, E), jnp.int32),
        ],
    )(logits)


@functools.cache
def _sc_mesh():
    return plsc.VectorSubcoreMesh(
        core_axis_name="c", subcore_axis_name="s",
        num_cores=NC, num_subcores=NS)


@functools.cache
def _sc_dispatch_kernel():
    return pl.kernel(
        _sc_dispatch_body,
        mesh=_sc_mesh(),
        out_type=jax.ShapeDtypeStruct((R, D), jnp.float32),
        scratch_types=[
            pltpu.VMEM((TPW, D), jnp.float32),
            pltpu.VMEM((TPW,), jnp.int32),
            pltpu.VMEM((TPW,), jnp.int32),
            pltpu.SemaphoreType.DMA,
        ],
    )


def _sc_dispatch_body(x1_hbm, pos_hbm, xg_hbm,
                      rows_v, idx0_v, idx1_v, sem):
    wid = lax.axis_index("s") * NC + lax.axis_index("c")
    base = wid * TPW
    pltpu.sync_copy(x1_hbm.at[pl.ds(base, TPW)], rows_v)
    pltpu.sync_copy(pos_hbm.at[pl.ds(base, TPW)], idx0_v)
    pltpu.sync_copy(pos_hbm.at[pl.ds(N + base, TPW)], idx1_v)
    pltpu.async_copy(rows_v, xg_hbm.at[idx0_v], sem).wait()
    pltpu.async_copy(rows_v, xg_hbm.at[idx1_v], sem).wait()


DB = 512
FB2 = 2048
NPA = D // DB
NPB = FF // FB2
HMAX = 512


def _ffn_body(nb_ref, off_ref, xg_ref, w1_ref, b1_ref, w2_ref, b2_ref, s_ref,
              yg_ref, h_ref):
    e = pl.program_id(0)
    p = pl.program_id(1)
    nb_e = nb_ref[e]
    off_e = off_ref[e]

    @pl.when(p < NPA)
    def _phase_a():
        d0 = pl.multiple_of(p * DB, DB)

        def body_first(i, carry):
            r0 = pl.multiple_of(off_e + i * M, M)
            l0 = pl.multiple_of(i * M, M)
            xb = xg_ref[pl.ds(r0, M), pl.ds(d0, DB)]
            acc = jnp.dot(xb, w1_ref[0], preferred_element_type=jnp.float32,
                          precision=lax.Precision.DEFAULT)
            b1b = jnp.broadcast_to(b1_ref[0], acc.shape)
            h_ref[pl.ds(l0, M), :] = (acc + b1b).astype(jnp.bfloat16)
            return carry

        def body_rest(i, carry):
            r0 = pl.multiple_of(off_e + i * M, M)
            l0 = pl.multiple_of(i * M, M)
            xb = xg_ref[pl.ds(r0, M), pl.ds(d0, DB)]
            acc = jnp.dot(xb, w1_ref[0], preferred_element_type=jnp.float32,
                          precision=lax.Precision.DEFAULT)
            prev = h_ref[pl.ds(l0, M), :].astype(jnp.float32)
            h_ref[pl.ds(l0, M), :] = (acc + prev).astype(jnp.bfloat16)
            return carry

        @pl.when(p == 0)
        def _():
            lax.fori_loop(0, nb_e, body_first, 0)

        @pl.when(p != 0)
        def _():
            lax.fori_loop(0, nb_e, body_rest, 0)

    @pl.when(p >= NPA)
    def _phase_b():
        f0 = pl.multiple_of((p - NPA) * FB2, FB2)

        def body_first(i, carry):
            r0 = pl.multiple_of(off_e + i * M, M)
            l0 = pl.multiple_of(i * M, M)
            hb = jnp.maximum(h_ref[pl.ds(l0, M), pl.ds(f0, FB2)], 0.0)
            contrib = jnp.dot(hb.astype(jnp.float32), w2_ref[0],
                              preferred_element_type=jnp.float32,
                              precision=lax.Precision.DEFAULT)
            b2b = jnp.broadcast_to(b2_ref[0], contrib.shape)
            yg_ref[pl.ds(r0, M), :] = contrib + b2b
            return carry

        def body_last(i, carry):
            r0 = pl.multiple_of(off_e + i * M, M)
            l0 = pl.multiple_of(i * M, M)
            hb = jnp.maximum(h_ref[pl.ds(l0, M), pl.ds(f0, FB2)], 0.0)
            contrib = jnp.dot(hb.astype(jnp.float32), w2_ref[0],
                              preferred_element_type=jnp.float32,
                              precision=lax.Precision.DEFAULT)
            prev = yg_ref[pl.ds(r0, M), :]
            sb = s_ref[pl.ds(r0, M), :]
            yg_ref[pl.ds(r0, M), :] = (contrib + prev) * sb
            return carry

        @pl.when(p == NPA)
        def _():
            lax.fori_loop(0, nb_e, body_first, 0)

        @pl.when(p == NPA + NPB - 1)
        def _():
            lax.fori_loop(0, nb_e, body_last, 0)


def _ffn_call(nb, off, xg, e_w1, e_b1, e_w2, e_b2, sslot2d):
    grid_spec = pltpu.PrefetchScalarGridSpec(
        num_scalar_prefetch=2,
        grid=(E, NPA + NPB),
        in_specs=[
            pl.BlockSpec((R, D), lambda e, p, nb, off: (0, 0)),
            pl.BlockSpec((1, DB, FF),
                         lambda e, p, nb, off: (e, jnp.minimum(p, NPA - 1), 0)),
            pl.BlockSpec((1, 1, FF), lambda e, p, nb, off: (e, 0, 0)),
            pl.BlockSpec((1, FB2, D),
                         lambda e, p, nb, off: (e, jnp.maximum(p - NPA, 0), 0)),
            pl.BlockSpec((1, 1, D), lambda e, p, nb, off: (e, 0, 0)),
            pl.BlockSpec((R, 1), lambda e, p, nb, off: (0, 0)),
        ],
        out_specs=pl.BlockSpec((R, D), lambda e, p, nb, off: (0, 0)),
        scratch_shapes=[pltpu.VMEM((HMAX, FF), jnp.bfloat16)],
    )
    return pl.pallas_call(
        _ffn_body,
        grid_spec=grid_spec,
        out_shape=jax.ShapeDtypeStruct((R, D), jnp.float32),
    )(nb, off, xg, e_w1, e_b1.reshape(E, 1, FF), e_w2, e_b2.reshape(E, 1, D),
      sslot2d)


@functools.cache
def _sc_combine_kernel():
    return pl.kernel(
        _sc_combine_body,
        mesh=_sc_mesh(),
        out_type=jax.ShapeDtypeStruct((N, D), jnp.float32),
        scratch_types=[
            pltpu.VMEM((TPW, D), jnp.float32),
            pltpu.VMEM((TPW, D), jnp.float32),
            pltpu.VMEM((TPW, D), jnp.float32),
            pltpu.VMEM((TPW,), jnp.int32),
            pltpu.VMEM((TPW,), jnp.int32),
            pltpu.SemaphoreType.DMA,
        ],
    )


def _sc_combine_body(yg_hbm, pos_hbm, x1_hbm, res_hbm,
                     r0_v, r1_v, rx_v, idx0_v, idx1_v, sem):
    wid = lax.axis_index("s") * NC + lax.axis_index("c")
    base = wid * TPW
    pltpu.sync_copy(pos_hbm.at[pl.ds(base, TPW)], idx0_v)
    pltpu.sync_copy(pos_hbm.at[pl.ds(N + base, TPW)], idx1_v)
    pltpu.async_copy(yg_hbm.at[idx0_v], r0_v, sem).wait()
    pltpu.async_copy(yg_hbm.at[idx1_v], r1_v, sem).wait()
    pltpu.sync_copy(x1_hbm.at[pl.ds(base, TPW)], rx_v)

    UNROLL = 4
    CH = D // (UNROLL * L)

    def body(i, carry):
        t = i // CH
        d = i % CH
        for u in range(UNROLL):
            sl = pl.ds(d * UNROLL * L + u * L, L)
            r0_v[t, sl] = r0_v[t, sl] + r1_v[t, sl] + rx_v[t, sl]
        return carry

    lax.fori_loop(0, TPW * CH, body, 0)
    pltpu.sync_copy(r0_v, res_hbm.at[pl.ds(base, TPW)])


def _ln2_body(res_ref, g_ref, b_ref, out_ref):
    xb = res_ref[...]
    m = jnp.mean(xb, axis=1, keepdims=True)
    c = xb - m
    v = jnp.mean(c * c, axis=1, keepdims=True)
    out_ref[...] = c * lax.rsqrt(v + EPS) * g_ref[...] + b_ref[...]


def _ln2_call(res, g2, b2):
    return pl.pallas_call(
        _ln2_body,
        grid=(N // M,),
        in_specs=[
            pl.BlockSpec((M, D), lambda i: (i, 0)),
            pl.BlockSpec((1, D), lambda i: (0, 0)),
            pl.BlockSpec((1, D), lambda i: (0, 0)),
        ],
        out_specs=pl.BlockSpec((M, D), lambda i: (i, 0)),
        out_shape=jax.ShapeDtypeStruct((N, D), jnp.float32),
    )(res, g2, b2)


def kernel(x, fourier_bias, key_w, key_b, value_w, value_b, out_w, out_b,
           gate_w, gate_b, e_w1, e_b1, e_w2, e_b2, ln1_g, ln1_b, ln2_g, ln2_b):
    row = lambda v: v.reshape(1, -1)
    x1, logits = _attn_call(
        x, fourier_bias, key_w, row(key_b), value_w, row(value_b),
        out_w, row(out_b), gate_w, row(gate_b), row(ln1_g), row(ln1_b))
    x1_2d = x1.reshape(N, D)
    pos, sslot, meta = _route_call(logits.reshape(N, E))
    pos1 = pos.reshape(NA)
    xg = _sc_dispatch_kernel()(x1_2d, pos1)
    yg = _ffn_call(meta[0], meta[1], xg, e_w1, e_b1, e_w2, e_b2,
                   sslot.reshape(R, 1))
    res = _sc_combine_kernel()(yg, pos1, x1_2d)
    out = _ln2_call(res, row(ln2_g), row(ln2_b))
    return out.reshape(B, T, D)


# speedup vs baseline: 1.0459x; 1.0052x over previous
import functools

import jax
import jax.numpy as jnp
from jax import lax
from jax.experimental import pallas as pl
from jax.experimental.pallas import tpu as pltpu
from jax.experimental.pallas import tpu_sc as plsc

B, T, D = 2, 256, 1024
H = 16
FF = 4096
E = 8
EPS = 1e-5
N = B * T
NA = 2 * N
M = 128
R = 2048
FB = 512
NC, NS, L = 2, 16, 16
NW = NC * NS
TPW = N // NW


def _attn_body(x_ref, fb_ref, kw_ref, kb_ref, vw_ref, vb_ref, ow_ref, ob_ref,
               gw_ref, gb_ref, g1_ref, b1_ref, x1_ref, logits_ref):
    xb = x_ref[0]
    K = jnp.dot(xb, kw_ref[...], preferred_element_type=jnp.float32) + kb_ref[...]
    V = jnp.dot(xb, vw_ref[...], preferred_element_type=jnp.float32) + vb_ref[...]
    term1 = jnp.sum(K * V, axis=0, keepdims=True)
    wv = jnp.dot(fb_ref[0], V, preferred_element_type=jnp.float32) + term1
    attn = jnp.dot(wv, ow_ref[...], preferred_element_type=jnp.float32) + ob_ref[...]
    pre = xb + attn
    m = jnp.mean(pre, axis=1, keepdims=True)
    c = pre - m
    v = jnp.mean(c * c, axis=1, keepdims=True)
    x1 = c * lax.rsqrt(v + EPS) * g1_ref[...] + b1_ref[...]
    x1_ref[0] = x1
    logits_ref[0] = jnp.dot(x1, gw_ref[...], preferred_element_type=jnp.float32) + gb_ref[...]


def _attn_call(x, fb, kw, kb, vw, vb, ow, ob, gw, gb, g1, b1):
    full2 = lambda b: (0, 0)
    return pl.pallas_call(
        _attn_body,
        grid=(B,),
        in_specs=[
            pl.BlockSpec((1, T, D), lambda b: (b, 0, 0)),
            pl.BlockSpec((1, T, T), lambda b: (b, 0, 0)),
            pl.BlockSpec((D, D), full2),
            pl.BlockSpec((1, D), full2),
            pl.BlockSpec((D, D), full2),
            pl.BlockSpec((1, D), full2),
            pl.BlockSpec((D, D), full2),
            pl.BlockSpec((1, D), full2),
            pl.BlockSpec((D, E), full2),
            pl.BlockSpec((1, E), full2),
            pl.BlockSpec((1, D), full2),
            pl.BlockSpec((1, D), full2),
        ],
        out_specs=[
            pl.BlockSpec((1, T, D), lambda b: (b, 0, 0)),
            pl.BlockSpec((1, T, E), lambda b: (b, 0, 0)),
        ],
        out_shape=[
            jax.ShapeDtypeStruct((B, T, D), jnp.float32),
            jax.ShapeDtypeStruct((B, T, E), jnp.float32),
        ],
    )(x, fb, kw, kb, vw, vb, ow, ob, gw, gb, g1, b1)


def _route_body(logits_ref, pos_ref, sslot_ref, meta_ref):
    lg = logits_ref[...]
    mx = jnp.max(lg, axis=1, keepdims=True)
    ex = jnp.exp(lg - mx)
    sc = ex / jnp.sum(ex, axis=1, keepdims=True)
    io = lax.broadcasted_iota(jnp.int32, (N, E), 1)
    m1 = jnp.max(sc, axis=1, keepdims=True)
    e1 = jnp.min(jnp.where(sc == m1, io, E), axis=1, keepdims=True)
    sc2 = jnp.where(io == e1, -1.0, sc)
    m2 = jnp.max(sc2, axis=1, keepdims=True)
    e2 = jnp.min(jnp.where(sc2 == m2, io, E), axis=1, keepdims=True)
    ef = jnp.concatenate([e1, e2], axis=0)
    sf = jnp.concatenate([m1, m2], axis=0)
    ioA = lax.broadcasted_iota(jnp.int32, (NA, E), 1)
    oh = (ioA == ef).astype(jnp.float32)
    ri = lax.broadcasted_iota(jnp.int32, (NA, NA), 0)
    ci = lax.broadcasted_iota(jnp.int32, (NA, NA), 1)
    tril = (ci <= ri).astype(jnp.float32)
    cum = jnp.dot(tril, oh, preferred_element_type=jnp.float32)
    rank = jnp.sum(cum * oh, axis=1, keepdims=True) - 1.0
    counts = jnp.sum(oh, axis=0, keepdims=True)
    nbf = jnp.floor((counts + (M - 1)) / M)
    ui = lax.broadcasted_iota(jnp.int32, (E, E), 0)
    uj = lax.broadcasted_iota(jnp.int32, (E, E), 1)
    ux = (ui < uj).astype(jnp.float32)
    excl = jnp.dot(nbf, ux, preferred_element_type=jnp.float32)
    off = excl * M
    offsel = jnp.sum(oh * off, axis=1, keepdims=True)
    posi = (offsel + rank).astype(jnp.int32)
    pos_ref[...] = posi
    ioR = lax.broadcasted_iota(jnp.int32, (NA, R), 1)
    sslot_ref[...] = jnp.sum(jnp.where(ioR == posi, sf, 0.0), axis=0,
                             keepdims=True)
    meta_ref[...] = jnp.concatenate([nbf, off], axis=0).astype(jnp.int32)


def _route_call(logits):
    return pl.pallas_call(
        _route_body,
        out_shape=[
            jax.ShapeDtypeStruct((NA, 1), jnp.int32),
            jax.ShapeDtypeStruct((1, R), jnp.float32),
            jax.ShapeDtypeStruct((2, E), jnp.int32),
        ],
    )(logits)


@functools.cache
def _sc_mesh():
    return plsc.VectorSubcoreMesh(
        core_axis_name="c", subcore_axis_name="s",
        num_cores=NC, num_subcores=NS)


@functools.cache
def _sc_dispatch_kernel():
    return pl.kernel(
        _sc_dispatch_body,
        mesh=_sc_mesh(),
        out_type=jax.ShapeDtypeStruct((R, D), jnp.float32),
        scratch_types=[
            pltpu.VMEM((TPW, D), jnp.float32),
            pltpu.VMEM((TPW,), jnp.int32),
            pltpu.VMEM((TPW,), jnp.int32),
            pltpu.SemaphoreType.DMA,
        ],
    )


def _sc_dispatch_body(x1_hbm, pos_hbm, xg_hbm,
                      rows_v, idx0_v, idx1_v, sem):
    wid = lax.axis_index("s") * NC + lax.axis_index("c")
    base = wid * TPW
    pltpu.sync_copy(x1_hbm.at[pl.ds(base, TPW)], rows_v)
    pltpu.sync_copy(pos_hbm.at[pl.ds(base, TPW)], idx0_v)
    pltpu.sync_copy(pos_hbm.at[pl.ds(N + base, TPW)], idx1_v)
    c0 = pltpu.async_copy(rows_v, xg_hbm.at[idx0_v], sem)
    c1 = pltpu.async_copy(rows_v, xg_hbm.at[idx1_v], sem)
    c0.wait()
    c1.wait()


DB = 512
FB2 = 2048
NPA = D // DB
NPB = FF // FB2
HMAX = 512


def _ffn_body(nb_ref, off_ref, xg_ref, w1_ref, b1_ref, w2_ref, b2_ref, s_ref,
              yg_ref, h_ref):
    e = pl.program_id(0)
    p = pl.program_id(1)
    nb_e = nb_ref[e]
    off_e = off_ref[e]

    @pl.when(p < NPA)
    def _phase_a():
        d0 = pl.multiple_of(p * DB, DB)

        def body_first(i, carry):
            r0 = pl.multiple_of(off_e + i * M, M)
            l0 = pl.multiple_of(i * M, M)
            xb = xg_ref[pl.ds(r0, M), pl.ds(d0, DB)]
            acc = jnp.dot(xb, w1_ref[0], preferred_element_type=jnp.float32,
                          precision=lax.Precision.DEFAULT)
            b1b = jnp.broadcast_to(b1_ref[0], acc.shape)
            h_ref[pl.ds(l0, M), :] = (acc + b1b).astype(jnp.bfloat16)
            return carry

        def body_rest(i, carry):
            r0 = pl.multiple_of(off_e + i * M, M)
            l0 = pl.multiple_of(i * M, M)
            xb = xg_ref[pl.ds(r0, M), pl.ds(d0, DB)]
            acc = jnp.dot(xb, w1_ref[0], preferred_element_type=jnp.float32,
                          precision=lax.Precision.DEFAULT)
            prev = h_ref[pl.ds(l0, M), :].astype(jnp.float32)
            h_ref[pl.ds(l0, M), :] = (acc + prev).astype(jnp.bfloat16)
            return carry

        @pl.when(p == 0)
        def _():
            lax.fori_loop(0, nb_e, body_first, 0)

        @pl.when(p != 0)
        def _():
            lax.fori_loop(0, nb_e, body_rest, 0)

    @pl.when(p >= NPA)
    def _phase_b():
        f0 = pl.multiple_of((p - NPA) * FB2, FB2)

        def body_first(i, carry):
            r0 = pl.multiple_of(off_e + i * M, M)
            l0 = pl.multiple_of(i * M, M)
            hb = jnp.maximum(h_ref[pl.ds(l0, M), pl.ds(f0, FB2)], 0.0)
            contrib = jnp.dot(hb.astype(jnp.float32), w2_ref[0],
                              preferred_element_type=jnp.float32,
                              precision=lax.Precision.DEFAULT)
            b2b = jnp.broadcast_to(b2_ref[0], contrib.shape)
            yg_ref[pl.ds(r0, M), :] = contrib + b2b
            return carry

        def body_last(i, carry):
            r0 = pl.multiple_of(off_e + i * M, M)
            l0 = pl.multiple_of(i * M, M)
            hb = jnp.maximum(h_ref[pl.ds(l0, M), pl.ds(f0, FB2)], 0.0)
            contrib = jnp.dot(hb.astype(jnp.float32), w2_ref[0],
                              preferred_element_type=jnp.float32,
                              precision=lax.Precision.DEFAULT)
            prev = yg_ref[pl.ds(r0, M), :]
            sb = s_ref[pl.ds(r0, M), :]
            yg_ref[pl.ds(r0, M), :] = (contrib + prev) * sb
            return carry

        @pl.when(p == NPA)
        def _():
            lax.fori_loop(0, nb_e, body_first, 0)

        @pl.when(p == NPA + NPB - 1)
        def _():
            lax.fori_loop(0, nb_e, body_last, 0)


def _ffn_call(nb, off, xg, e_w1, e_b1, e_w2, e_b2, sslot2d):
    grid_spec = pltpu.PrefetchScalarGridSpec(
        num_scalar_prefetch=2,
        grid=(E, NPA + NPB),
        in_specs=[
            pl.BlockSpec((R, D), lambda e, p, nb, off: (0, 0)),
            pl.BlockSpec((1, DB, FF),
                         lambda e, p, nb, off: (e, jnp.minimum(p, NPA - 1), 0)),
            pl.BlockSpec((1, 1, FF), lambda e, p, nb, off: (e, 0, 0)),
            pl.BlockSpec((1, FB2, D),
                         lambda e, p, nb, off: (e, jnp.maximum(p - NPA, 0), 0)),
            pl.BlockSpec((1, 1, D), lambda e, p, nb, off: (e, 0, 0)),
            pl.BlockSpec((R, 1), lambda e, p, nb, off: (0, 0)),
        ],
        out_specs=pl.BlockSpec((R, D), lambda e, p, nb, off: (0, 0)),
        scratch_shapes=[pltpu.VMEM((HMAX, FF), jnp.bfloat16)],
    )
    return pl.pallas_call(
        _ffn_body,
        grid_spec=grid_spec,
        out_shape=jax.ShapeDtypeStruct((R, D), jnp.float32),
    )(nb, off, xg, e_w1, e_b1.reshape(E, 1, FF), e_w2, e_b2.reshape(E, 1, D),
      sslot2d)


@functools.cache
def _sc_combine_kernel():
    return pl.kernel(
        _sc_combine_body,
        mesh=_sc_mesh(),
        out_type=jax.ShapeDtypeStruct((N, D), jnp.float32),
        scratch_types=[
            pltpu.VMEM((TPW, D), jnp.float32),
            pltpu.VMEM((TPW, D), jnp.float32),
            pltpu.VMEM((TPW, D), jnp.float32),
            pltpu.VMEM((TPW,), jnp.int32),
            pltpu.VMEM((TPW,), jnp.int32),
            pltpu.SemaphoreType.DMA,
            pltpu.SemaphoreType.DMA,
        ],
    )


def _sc_combine_body(yg_hbm, pos_hbm, x1_hbm, res_hbm,
                     r0_v, r1_v, rx_v, idx0_v, idx1_v, sem, sem2):
    wid = lax.axis_index("s") * NC + lax.axis_index("c")
    base = wid * TPW
    pltpu.sync_copy(pos_hbm.at[pl.ds(base, TPW)], idx0_v)
    pltpu.sync_copy(pos_hbm.at[pl.ds(N + base, TPW)], idx1_v)
    g0 = pltpu.async_copy(yg_hbm.at[idx0_v], r0_v, sem)
    g1 = pltpu.async_copy(yg_hbm.at[idx1_v], r1_v, sem)
    gx = pltpu.async_copy(x1_hbm.at[pl.ds(base, TPW)], rx_v, sem2)
    g0.wait()
    g1.wait()
    gx.wait()

    UNROLL = 4
    CH = D // (UNROLL * L)

    def body(i, carry):
        t = i // CH
        d = i % CH
        for u in range(UNROLL):
            sl = pl.ds(d * UNROLL * L + u * L, L)
            r0_v[t, sl] = r0_v[t, sl] + r1_v[t, sl] + rx_v[t, sl]
        return carry

    lax.fori_loop(0, TPW * CH, body, 0)
    pltpu.sync_copy(r0_v, res_hbm.at[pl.ds(base, TPW)])


def _ln2_body(res_ref, g_ref, b_ref, out_ref):
    xb = res_ref[...]
    m = jnp.mean(xb, axis=1, keepdims=True)
    c = xb - m
    v = jnp.mean(c * c, axis=1, keepdims=True)
    out_ref[...] = c * lax.rsqrt(v + EPS) * g_ref[...] + b_ref[...]


def _ln2_call(res, g2, b2):
    return pl.pallas_call(
        _ln2_body,
        grid=(N // M,),
        in_specs=[
            pl.BlockSpec((M, D), lambda i: (i, 0)),
            pl.BlockSpec((1, D), lambda i: (0, 0)),
            pl.BlockSpec((1, D), lambda i: (0, 0)),
        ],
        out_specs=pl.BlockSpec((M, D), lambda i: (i, 0)),
        out_shape=jax.ShapeDtypeStruct((N, D), jnp.float32),
    )(res, g2, b2)


def kernel(x, fourier_bias, key_w, key_b, value_w, value_b, out_w, out_b,
           gate_w, gate_b, e_w1, e_b1, e_w2, e_b2, ln1_g, ln1_b, ln2_g, ln2_b):
    row = lambda v: v.reshape(1, -1)
    x1, logits = _attn_call(
        x, fourier_bias, key_w, row(key_b), value_w, row(value_b),
        out_w, row(out_b), gate_w, row(gate_b), row(ln1_g), row(ln1_b))
    x1_2d = x1.reshape(N, D)
    pos, sslot, meta = _route_call(logits.reshape(N, E))
    pos1 = pos.reshape(NA)
    xg = _sc_dispatch_kernel()(x1_2d, pos1)
    yg = _ffn_call(meta[0], meta[1], xg, e_w1, e_b1, e_w2, e_b2,
                   sslot.reshape(R, 1))
    res = _sc_combine_kernel()(yg, pos1, x1_2d)
    out = _ln2_call(res, row(ln2_g), row(ln2_b))
    return out.reshape(B, T, D)


# speedup vs baseline: 1.0475x; 1.0016x over previous
import functools

import jax
import jax.numpy as jnp
from jax import lax
from jax.experimental import pallas as pl
from jax.experimental.pallas import tpu as pltpu
from jax.experimental.pallas import tpu_sc as plsc

B, T, D = 2, 256, 1024
H = 16
FF = 4096
E = 8
EPS = 1e-5
N = B * T
NA = 2 * N
M = 128
R = 2048
NC, NS, L = 2, 16, 16
NW = NC * NS
TPW = N // NW


def _attn_body(x_ref, fb_ref, kw_ref, kb_ref, vw_ref, vb_ref, ow_ref, ob_ref,
               gw_ref, gb_ref, g1_ref, b1_ref, x1_ref, logits_ref):
    xb = x_ref[0]
    K = jnp.dot(xb, kw_ref[...], preferred_element_type=jnp.float32) + kb_ref[...]
    V = jnp.dot(xb, vw_ref[...], preferred_element_type=jnp.float32) + vb_ref[...]
    term1 = jnp.sum(K * V, axis=0, keepdims=True)
    wv = jnp.dot(fb_ref[0], V, preferred_element_type=jnp.float32) + term1
    attn = jnp.dot(wv, ow_ref[...], preferred_element_type=jnp.float32) + ob_ref[...]
    pre = xb + attn
    m = jnp.mean(pre, axis=1, keepdims=True)
    c = pre - m
    v = jnp.mean(c * c, axis=1, keepdims=True)
    x1 = c * lax.rsqrt(v + EPS) * g1_ref[...] + b1_ref[...]
    x1_ref[0] = x1
    logits_ref[0] = jnp.dot(x1, gw_ref[...], preferred_element_type=jnp.float32) + gb_ref[...]


def _attn_call(x, fb, kw, kb, vw, vb, ow, ob, gw, gb, g1, b1):
    full2 = lambda b: (0, 0)
    return pl.pallas_call(
        _attn_body,
        grid=(B,),
        in_specs=[
            pl.BlockSpec((1, T, D), lambda b: (b, 0, 0)),
            pl.BlockSpec((1, T, T), lambda b: (b, 0, 0)),
            pl.BlockSpec((D, D), full2),
            pl.BlockSpec((1, D), full2),
            pl.BlockSpec((D, D), full2),
            pl.BlockSpec((1, D), full2),
            pl.BlockSpec((D, D), full2),
            pl.BlockSpec((1, D), full2),
            pl.BlockSpec((D, E), full2),
            pl.BlockSpec((1, E), full2),
            pl.BlockSpec((1, D), full2),
            pl.BlockSpec((1, D), full2),
        ],
        out_specs=[
            pl.BlockSpec((1, T, D), lambda b: (b, 0, 0)),
            pl.BlockSpec((1, T, E), lambda b: (b, 0, 0)),
        ],
        out_shape=[
            jax.ShapeDtypeStruct((B, T, D), jnp.float32),
            jax.ShapeDtypeStruct((B, T, E), jnp.float32),
        ],
    )(x, fb, kw, kb, vw, vb, ow, ob, gw, gb, g1, b1)


def _route_body(logits_ref, pos_ref, sslot_ref, meta_ref):
    lg = logits_ref[...]
    mx = jnp.max(lg, axis=1, keepdims=True)
    ex = jnp.exp(lg - mx)
    sc = ex / jnp.sum(ex, axis=1, keepdims=True)
    io = lax.broadcasted_iota(jnp.int32, (N, E), 1)
    m1 = jnp.max(sc, axis=1, keepdims=True)
    e1 = jnp.min(jnp.where(sc == m1, io, E), axis=1, keepdims=True)
    sc2 = jnp.where(io == e1, -1.0, sc)
    m2 = jnp.max(sc2, axis=1, keepdims=True)
    e2 = jnp.min(jnp.where(sc2 == m2, io, E), axis=1, keepdims=True)
    ef = jnp.concatenate([e1, e2], axis=0)
    sf = jnp.concatenate([m1, m2], axis=0)
    ioA = lax.broadcasted_iota(jnp.int32, (NA, E), 1)
    oh = (ioA == ef).astype(jnp.float32)
    ri = lax.broadcasted_iota(jnp.int32, (NA, NA), 0)
    ci = lax.broadcasted_iota(jnp.int32, (NA, NA), 1)
    tril = (ci <= ri).astype(jnp.float32)
    cum = jnp.dot(tril, oh, preferred_element_type=jnp.float32)
    rank = jnp.sum(cum * oh, axis=1, keepdims=True) - 1.0
    counts = jnp.sum(oh, axis=0, keepdims=True)
    nbf = jnp.floor((counts + (M - 1)) / M)
    ui = lax.broadcasted_iota(jnp.int32, (E, E), 0)
    uj = lax.broadcasted_iota(jnp.int32, (E, E), 1)
    ux = (ui < uj).astype(jnp.float32)
    excl = jnp.dot(nbf, ux, preferred_element_type=jnp.float32)
    off = excl * M
    offsel = jnp.sum(oh * off, axis=1, keepdims=True)
    posi = (offsel + rank).astype(jnp.int32)
    pos_ref[...] = posi
    ioR = lax.broadcasted_iota(jnp.int32, (NA, R), 1)
    sslot_ref[...] = jnp.sum(jnp.where(ioR == posi, sf, 0.0), axis=0,
                             keepdims=True)
    meta_ref[...] = jnp.concatenate([nbf, off], axis=0).astype(jnp.int32)


def _route_call(logits):
    return pl.pallas_call(
        _route_body,
        out_shape=[
            jax.ShapeDtypeStruct((NA, 1), jnp.int32),
            jax.ShapeDtypeStruct((1, R), jnp.float32),
            jax.ShapeDtypeStruct((2, E), jnp.int32),
        ],
    )(logits)


@functools.cache
def _sc_mesh():
    return plsc.VectorSubcoreMesh(
        core_axis_name="c", subcore_axis_name="s",
        num_cores=NC, num_subcores=NS)


@functools.cache
def _sc_dispatch_kernel():
    return pl.kernel(
        _sc_dispatch_body,
        mesh=_sc_mesh(),
        out_type=jax.ShapeDtypeStruct((R, D), jnp.float32),
        scratch_types=[
            pltpu.VMEM((TPW, D), jnp.float32),
            pltpu.VMEM((TPW,), jnp.int32),
            pltpu.VMEM((TPW,), jnp.int32),
            pltpu.SemaphoreType.DMA,
        ],
    )


def _sc_dispatch_body(x1_hbm, pos_hbm, xg_hbm,
                      rows_v, idx0_v, idx1_v, sem):
    wid = lax.axis_index("s") * NC + lax.axis_index("c")
    base = wid * TPW
    pltpu.sync_copy(x1_hbm.at[pl.ds(base, TPW)], rows_v)
    pltpu.sync_copy(pos_hbm.at[pl.ds(base, TPW)], idx0_v)
    pltpu.sync_copy(pos_hbm.at[pl.ds(N + base, TPW)], idx1_v)
    c0 = pltpu.async_copy(rows_v, xg_hbm.at[idx0_v], sem)
    c1 = pltpu.async_copy(rows_v, xg_hbm.at[idx1_v], sem)
    c0.wait()
    c1.wait()


DB = 512
FB2 = 2048
NPA = D // DB
NPB = FF // FB2
HMAX = 512


def _ffn_body(nb_ref, off_ref, xg_ref, w1_ref, b1_ref, w2_ref, b2_ref, s_ref,
              yg_ref, h_ref):
    e = pl.program_id(0)
    p = pl.program_id(1)
    nb_e = nb_ref[e]
    off_e = off_ref[e]

    @pl.when(p < NPA)
    def _phase_a():
        d0 = pl.multiple_of(p * DB, DB)

        def body_first(i, carry):
            r0 = pl.multiple_of(off_e + i * M, M)
            l0 = pl.multiple_of(i * M, M)
            xb = xg_ref[pl.ds(r0, M), pl.ds(d0, DB)]
            acc = jnp.dot(xb, w1_ref[0], preferred_element_type=jnp.float32,
                          precision=lax.Precision.DEFAULT)
            b1b = jnp.broadcast_to(b1_ref[0], acc.shape)
            h_ref[pl.ds(l0, M), :] = (acc + b1b).astype(jnp.bfloat16)
            return carry

        def body_rest(i, carry):
            r0 = pl.multiple_of(off_e + i * M, M)
            l0 = pl.multiple_of(i * M, M)
            xb = xg_ref[pl.ds(r0, M), pl.ds(d0, DB)]
            acc = jnp.dot(xb, w1_ref[0], preferred_element_type=jnp.float32,
                          precision=lax.Precision.DEFAULT)
            prev = h_ref[pl.ds(l0, M), :].astype(jnp.float32)
            h_ref[pl.ds(l0, M), :] = (acc + prev).astype(jnp.bfloat16)
            return carry

        @pl.when(p == 0)
        def _():
            lax.fori_loop(0, nb_e, body_first, 0)

        @pl.when(p != 0)
        def _():
            lax.fori_loop(0, nb_e, body_rest, 0)

    @pl.when(p >= NPA)
    def _phase_b():
        f0 = pl.multiple_of((p - NPA) * FB2, FB2)

        def body_first(i, carry):
            r0 = pl.multiple_of(off_e + i * M, M)
            l0 = pl.multiple_of(i * M, M)
            hb = jnp.maximum(h_ref[pl.ds(l0, M), pl.ds(f0, FB2)], 0.0)
            contrib = jnp.dot(hb.astype(jnp.float32), w2_ref[0],
                              preferred_element_type=jnp.float32,
                              precision=lax.Precision.DEFAULT)
            b2b = jnp.broadcast_to(b2_ref[0], contrib.shape)
            yg_ref[pl.ds(r0, M), :] = contrib + b2b
            return carry

        def body_last(i, carry):
            r0 = pl.multiple_of(off_e + i * M, M)
            l0 = pl.multiple_of(i * M, M)
            hb = jnp.maximum(h_ref[pl.ds(l0, M), pl.ds(f0, FB2)], 0.0)
            contrib = jnp.dot(hb.astype(jnp.float32), w2_ref[0],
                              preferred_element_type=jnp.float32,
                              precision=lax.Precision.DEFAULT)
            prev = yg_ref[pl.ds(r0, M), :]
            sb = s_ref[pl.ds(r0, M), :]
            yg_ref[pl.ds(r0, M), :] = (contrib + prev) * sb
            return carry

        @pl.when(p == NPA)
        def _():
            lax.fori_loop(0, nb_e, body_first, 0)

        @pl.when(p == NPA + NPB - 1)
        def _():
            lax.fori_loop(0, nb_e, body_last, 0)


def _ffn_call(nb, off, xg, e_w1, e_b1, e_w2, e_b2, sslot2d):
    grid_spec = pltpu.PrefetchScalarGridSpec(
        num_scalar_prefetch=2,
        grid=(E, NPA + NPB),
        in_specs=[
            pl.BlockSpec((R, D), lambda e, p, nb, off: (0, 0)),
            pl.BlockSpec((1, DB, FF),
                         lambda e, p, nb, off: (e, jnp.minimum(p, NPA - 1), 0)),
            pl.BlockSpec((1, 1, FF), lambda e, p, nb, off: (e, 0, 0)),
            pl.BlockSpec((1, FB2, D),
                         lambda e, p, nb, off: (e, jnp.maximum(p - NPA, 0), 0)),
            pl.BlockSpec((1, 1, D), lambda e, p, nb, off: (e, 0, 0)),
            pl.BlockSpec((R, 1), lambda e, p, nb, off: (0, 0)),
        ],
        out_specs=pl.BlockSpec((R, D), lambda e, p, nb, off: (0, 0)),
        scratch_shapes=[pltpu.VMEM((HMAX, FF), jnp.bfloat16)],
    )
    return pl.pallas_call(
        _ffn_body,
        grid_spec=grid_spec,
        out_shape=jax.ShapeDtypeStruct((R, D), jnp.float32),
    )(nb, off, xg, e_w1, e_b1.reshape(E, 1, FF), e_w2, e_b2.reshape(E, 1, D),
      sslot2d)


@functools.cache
def _sc_combine_kernel():
    return pl.kernel(
        _sc_combine_body,
        mesh=_sc_mesh(),
        out_type=jax.ShapeDtypeStruct((N, D), jnp.float32),
        scratch_types=[
            pltpu.VMEM((TPW, D), jnp.float32),
            pltpu.VMEM((TPW, D), jnp.float32),
            pltpu.VMEM((TPW, D), jnp.float32),
            pltpu.VMEM((TPW,), jnp.int32),
            pltpu.VMEM((TPW,), jnp.int32),
            pltpu.SemaphoreType.DMA,
            pltpu.SemaphoreType.DMA,
        ],
    )


def _sc_combine_body(yg_hbm, pos_hbm, x1_hbm, res_hbm,
                     r0_v, r1_v, rx_v, idx0_v, idx1_v, sem, sem2):
    wid = lax.axis_index("s") * NC + lax.axis_index("c")
    base = wid * TPW
    pltpu.sync_copy(pos_hbm.at[pl.ds(base, TPW)], idx0_v)
    pltpu.sync_copy(pos_hbm.at[pl.ds(N + base, TPW)], idx1_v)
    g0 = pltpu.async_copy(yg_hbm.at[idx0_v], r0_v, sem)
    g1 = pltpu.async_copy(yg_hbm.at[idx1_v], r1_v, sem)
    gx = pltpu.async_copy(x1_hbm.at[pl.ds(base, TPW)], rx_v, sem2)
    g0.wait()
    g1.wait()
    gx.wait()

    UNROLL = 4
    CH = D // (UNROLL * L)

    def body(i, carry):
        t = i // CH
        d = i % CH
        for u in range(UNROLL):
            sl = pl.ds(d * UNROLL * L + u * L, L)
            r0_v[t, sl] = r0_v[t, sl] + r1_v[t, sl] + rx_v[t, sl]
        return carry

    lax.fori_loop(0, TPW * CH, body, 0)
    pltpu.sync_copy(r0_v, res_hbm.at[pl.ds(base, TPW)])


def _ln2_body(res_ref, g_ref, b_ref, out_ref):
    xb = res_ref[...]
    m = jnp.mean(xb, axis=1, keepdims=True)
    c = xb - m
    v = jnp.mean(c * c, axis=1, keepdims=True)
    out_ref[...] = c * lax.rsqrt(v + EPS) * g_ref[...] + b_ref[...]


def _ln2_call(res, g2, b2):
    return pl.pallas_call(
        _ln2_body,
        grid=(N // M,),
        in_specs=[
            pl.BlockSpec((M, D), lambda i: (i, 0)),
            pl.BlockSpec((1, D), lambda i: (0, 0)),
            pl.BlockSpec((1, D), lambda i: (0, 0)),
        ],
        out_specs=pl.BlockSpec((M, D), lambda i: (i, 0)),
        out_shape=jax.ShapeDtypeStruct((N, D), jnp.float32),
    )(res, g2, b2)


def kernel(x, fourier_bias, key_w, key_b, value_w, value_b, out_w, out_b,
           gate_w, gate_b, e_w1, e_b1, e_w2, e_b2, ln1_g, ln1_b, ln2_g, ln2_b):
    row = lambda v: v.reshape(1, -1)
    x1, logits = _attn_call(
        x, fourier_bias, key_w, row(key_b), value_w, row(value_b),
        out_w, row(out_b), gate_w, row(gate_b), row(ln1_g), row(ln1_b))
    x1_2d = x1.reshape(N, D)
    pos, sslot, meta = _route_call(logits.reshape(N, E))
    pos1 = pos.reshape(NA)
    xg = _sc_dispatch_kernel()(x1_2d, pos1)
    yg = _ffn_call(meta[0], meta[1], xg, e_w1, e_b1, e_w2, e_b2,
                   sslot.reshape(R, 1))
    res = _sc_combine_kernel()(yg, pos1, x1_2d)
    out = _ln2_call(res, row(ln2_g), row(ln2_b))
    return out.reshape(B, T, D)
